# Initial kernel scaffold; baseline (speedup 1.0000x reference)
import jax
import jax.numpy as jnp
from jax.experimental import pallas as pl


def kernel(x, w9_0, b_0, w9_1, b_1, w9_2, b_2, w9_3, b_3, w9_4, b_4, w9_5, b_5, w9_6, b_6, w9_7, b_7, w9_8, b_8, w9_9, b_9, w9_10, b_10, w9_11, b_11, w9_12, b_12):
    raise NotImplementedError("write your pallas kernel here")


# speedup vs baseline: 1.7562x; 1.7562x over previous
import jax
import jax.numpy as jnp
from jax.experimental import pallas as pl
from jax.experimental.pallas import tpu as pltpu


def _conv_kernel(x_ref, w_ref, b_ref, o_ref):
    _, TH, W, TC = o_ref.shape
    cin = x_ref.shape[-1]
    row0 = pl.multiple_of(pl.program_id(2) * TH, TH)

    acc = jnp.zeros((TH * W, TC), dtype=jnp.float32)
    for k in range(9):
        dy, dx = k // 3, k % 3
        patch = x_ref[0, pl.ds(row0 + dy, TH), pl.ds(dx, W), :]
        acc = acc + jnp.dot(patch.reshape(TH * W, cin), w_ref[k],
                            preferred_element_type=jnp.float32)
    acc = acc + b_ref[...]
    o_ref[0] = jnp.maximum(acc, 0.0).reshape(TH, W, TC).astype(o_ref.dtype)


def _conv3x3_bn_relu(x_nhwc, w9, bias):
    B, H, W, Cin = x_nhwc.shape
    Cout = w9.shape[-1]
    x_pad = jnp.pad(x_nhwc, ((0, 0), (1, 1), (1, 1), (0, 0)))

    TC = min(Cout, 256)
    TH = H
    while TH * W > 512 or H % TH:
        TH -= 1
    grid = (B, Cout // TC, H // TH)

    return pl.pallas_call(
        _conv_kernel,
        grid=grid,
        in_specs=[
            pl.BlockSpec((1, H + 2, W + 2, Cin), lambda b, j, r: (b, 0, 0, 0)),
            pl.BlockSpec((9, Cin, TC), lambda b, j, r: (0, 0, j)),
            pl.BlockSpec((1, TC), lambda b, j, r: (0, j)),
        ],
        out_specs=pl.BlockSpec((1, TH, W, TC), lambda b, j, r: (b, r, 0, j)),
        out_shape=jax.ShapeDtypeStruct((B, H, W, Cout), jnp.bfloat16),
        compiler_params=pltpu.CompilerParams(
            dimension_semantics=("parallel", "parallel", "arbitrary")),
    )(x_pad, w9, bias)


def _maxpool_kernel(x_ref, o_ref):
    _, Ho, Wo, C = o_ref.shape
    x = x_ref[0]
    x = x.reshape(Ho, 2, Wo, 2 * C)
    m = jnp.max(x, axis=1)
    o_ref[0] = jnp.maximum(m[..., :C], m[..., C:])


def _maxpool2x2(x_nhwc):
    B, H, W, C = x_nhwc.shape
    Ho, Wo = H // 2, W // 2
    x_r = x_nhwc.reshape(B, H, Wo, 2 * C)
    return pl.pallas_call(
        _maxpool_kernel,
        grid=(B,),
        in_specs=[pl.BlockSpec((1, H, Wo, 2 * C), lambda b: (b, 0, 0, 0))],
        out_specs=pl.BlockSpec((1, Ho, Wo, C), lambda b: (b, 0, 0, 0)),
        out_shape=jax.ShapeDtypeStruct((B, Ho, Wo, C), jnp.bfloat16),
        compiler_params=pltpu.CompilerParams(dimension_semantics=("parallel",)),
    )(x_r)


_CFG = [64, 64, 'M', 128, 128, 'M', 256, 256, 256, 'M',
        512, 512, 512, 'M', 512, 512, 512, 'M']


def kernel(x, w9_0, b_0, w9_1, b_1, w9_2, b_2, w9_3, b_3, w9_4, b_4,
           w9_5, b_5, w9_6, b_6, w9_7, b_7, w9_8, b_8, w9_9, b_9,
           w9_10, b_10, w9_11, b_11, w9_12, b_12):
    ws = [w9_0, w9_1, w9_2, w9_3, w9_4, w9_5, w9_6, w9_7, w9_8, w9_9,
          w9_10, w9_11, w9_12]
    bs = [b_0, b_1, b_2, b_3, b_4, b_5, b_6, b_7, b_8, b_9, b_10, b_11, b_12]

    h = jnp.transpose(x, (0, 2, 3, 1)).astype(jnp.bfloat16)
    pooled = []
    li = 0
    for v in _CFG:
        if v == 'M':
            h = _maxpool2x2(h)
            pooled.append(h)
        else:
            h = _conv3x3_bn_relu(h, ws[li].astype(jnp.bfloat16), bs[li])
            li += 1
    return [jnp.transpose(o, (0, 3, 1, 2)).astype(jnp.float32)
            for o in pooled[1:]]


# speedup vs baseline: 2.2994x; 1.3094x over previous
import jax
import jax.numpy as jnp
from jax.experimental import pallas as pl
from jax.experimental.pallas import tpu as pltpu


def _relu_bias(acc, b):
    return jnp.maximum(acc + b, 0.0)


def _pool_write(acc, o_ref):
    _, THo, Wo, TC = o_ref.shape
    m = acc.reshape(THo * 2 * Wo, 2, TC)
    m = jnp.max(m, axis=1)
    m = m.reshape(THo, 2, Wo, TC)
    o_ref[0] = jnp.max(m, axis=1).astype(o_ref.dtype)


def _make_conv_kernel(merged, pooled):
    def body(x_ref, w_ref, b_ref, o_ref):
        TH = o_ref.shape[1] * (2 if pooled else 1)
        W = o_ref.shape[2] * (2 if pooled else 1)
        TC = o_ref.shape[3]
        cin = x_ref.shape[-1]
        row0 = pl.multiple_of(pl.program_id(2) * TH, TH)

        if merged:
            taps = []
            for k in range(9):
                dy, dx = k // 3, k % 3
                taps.append(x_ref[0, pl.ds(row0 + dy, TH), pl.ds(dx, W), :])
            patch = jnp.concatenate(taps, axis=-1).reshape(TH * W, 9 * cin)
            acc = jnp.dot(patch, w_ref[...],
                          preferred_element_type=jnp.float32)
        else:
            acc = jnp.zeros((TH * W, TC), dtype=jnp.float32)
            for k in range(9):
                dy, dx = k // 3, k % 3
                patch = x_ref[0, pl.ds(row0 + dy, TH), pl.ds(dx, W), :]
                acc = acc + jnp.dot(patch.reshape(TH * W, cin), w_ref[k],
                                    preferred_element_type=jnp.float32)
        acc = _relu_bias(acc, b_ref[...])
        if pooled:
            _pool_write(acc, o_ref)
        else:
            o_ref[0] = acc.reshape(TH, W, TC).astype(o_ref.dtype)
    return body


def _conv3x3(x_nhwc, w9, bias, pooled):
    B, H, W, Cin = x_nhwc.shape
    Cout = w9.shape[-1]
    x_pad = jnp.pad(x_nhwc, ((0, 0), (1, 1), (1, 1), (0, 0)))

    merged = Cin <= 128
    TC = min(Cout, 256)
    TH = H
    while TH * W > 1024 or H % TH or TH % 2:
        TH -= 1
    grid = (B, Cout // TC, H // TH)

    if merged:
        w_arg = w9.reshape(9 * Cin, Cout)
        w_spec = pl.BlockSpec((9 * Cin, TC), lambda b, j, r: (0, j))
    else:
        w_arg = w9
        w_spec = pl.BlockSpec((9, Cin, TC), lambda b, j, r: (0, 0, j))

    if pooled:
        out_shape = jax.ShapeDtypeStruct((B, H // 2, W // 2, Cout),
                                         jnp.bfloat16)
        out_spec = pl.BlockSpec((1, TH // 2, W // 2, TC),
                                lambda b, j, r: (b, r, 0, j))
    else:
        out_shape = jax.ShapeDtypeStruct((B, H, W, Cout), jnp.bfloat16)
        out_spec = pl.BlockSpec((1, TH, W, TC), lambda b, j, r: (b, r, 0, j))

    return pl.pallas_call(
        _make_conv_kernel(merged, pooled),
        grid=grid,
        in_specs=[
            pl.BlockSpec((1, H + 2, W + 2, Cin), lambda b, j, r: (b, 0, 0, 0)),
            w_spec,
            pl.BlockSpec((1, TC), lambda b, j, r: (0, j)),
        ],
        out_specs=out_spec,
        out_shape=out_shape,
        compiler_params=pltpu.CompilerParams(
            dimension_semantics=("parallel", "parallel", "arbitrary")),
    )(x_pad, w_arg, bias)


def kernel(x, w9_0, b_0, w9_1, b_1, w9_2, b_2, w9_3, b_3, w9_4, b_4,
           w9_5, b_5, w9_6, b_6, w9_7, b_7, w9_8, b_8, w9_9, b_9,
           w9_10, b_10, w9_11, b_11, w9_12, b_12):
    ws = [w9_0, w9_1, w9_2, w9_3, w9_4, w9_5, w9_6, w9_7, w9_8, w9_9,
          w9_10, w9_11, w9_12]
    bs = [b_0, b_1, b_2, b_3, b_4, b_5, b_6, b_7, b_8, b_9, b_10, b_11, b_12]

    h = jnp.transpose(x, (0, 2, 3, 1)).astype(jnp.bfloat16)
    pooled = []
    pool_after = {1, 3, 6, 9, 12}
    for i in range(13):
        h = _conv3x3(h, ws[i].astype(jnp.bfloat16), bs[i], i in pool_after)
        if i in pool_after:
            pooled.append(h)
    return [jnp.transpose(o, (0, 3, 1, 2)).astype(jnp.float32)
            for o in pooled[1:]]


# speedup vs baseline: 2.7568x; 1.1989x over previous
import jax
import jax.numpy as jnp
from jax.experimental import pallas as pl
from jax.experimental.pallas import tpu as pltpu


def _stem_kernel(p_ref, w_ref, b_ref, o_ref):
    _, TH, W, K = p_ref.shape
    TC = o_ref.shape[-1]
    acc = jnp.dot(p_ref[0].reshape(TH * W, K), w_ref[...],
                  preferred_element_type=jnp.float32)
    acc = jnp.maximum(acc + b_ref[...], 0.0)
    o_ref[0] = acc.reshape(TH, W, TC).astype(o_ref.dtype)


def _stem(x_nhwc, w9, bias):
    B, H, W, _ = x_nhwc.shape
    Cout = w9.shape[-1]
    x_pad = jnp.pad(x_nhwc, ((0, 0), (1, 1), (1, 1), (0, 0)))
    p = jnp.concatenate(
        [x_pad[:, dy:dy + H, dx:dx + W, :] for dy in range(3)
         for dx in range(3)], axis=-1).astype(jnp.bfloat16)
    TH = 8
    grid = (B, H // TH)
    return pl.pallas_call(
        _stem_kernel,
        grid=grid,
        in_specs=[
            pl.BlockSpec((1, TH, W, 27), lambda b, r: (b, r, 0, 0)),
            pl.BlockSpec((27, Cout), lambda b, r: (0, 0)),
            pl.BlockSpec((1, Cout), lambda b, r: (0, 0)),
        ],
        out_specs=pl.BlockSpec((1, TH, W, Cout), lambda b, r: (b, r, 0, 0)),
        out_shape=jax.ShapeDtypeStruct((B, H, W, Cout), jnp.bfloat16),
        compiler_params=pltpu.CompilerParams(
            dimension_semantics=("parallel", "arbitrary")),
    )(p, w9.reshape(27, Cout).astype(jnp.bfloat16), bias)


def _make_conv_kernel(merged, hpool):
    def body(x_ref, w_ref, b_ref, o_ref):
        TH = o_ref.shape[1] * (2 if hpool else 1)
        W = o_ref.shape[2]
        TC = o_ref.shape[3]
        cin = x_ref.shape[-1]
        row0 = pl.multiple_of(pl.program_id(2) * TH, TH)

        if merged:
            taps = []
            for k in range(9):
                dy, dx = k // 3, k % 3
                taps.append(x_ref[0, pl.ds(row0 + dy, TH), pl.ds(dx, W), :])
            patch = jnp.concatenate(taps, axis=-1).reshape(TH * W, 9 * cin)
            acc = jnp.dot(patch, w_ref[...],
                          preferred_element_type=jnp.float32)
        else:
            acc = jnp.zeros((TH * W, TC), dtype=jnp.float32)
            for k in range(9):
                dy, dx = k // 3, k % 3
                patch = x_ref[0, pl.ds(row0 + dy, TH), pl.ds(dx, W), :]
                acc = acc + jnp.dot(patch.reshape(TH * W, cin), w_ref[k],
                                    preferred_element_type=jnp.float32)
        acc = jnp.maximum(acc + b_ref[...], 0.0)
        if hpool:
            m = acc.reshape(TH // 2, 2, W, TC)
            o_ref[0] = jnp.max(m, axis=1).astype(o_ref.dtype)
        else:
            o_ref[0] = acc.reshape(TH, W, TC).astype(o_ref.dtype)
    return body


def _conv3x3(x_nhwc, w9, bias, hpool):
    B, H, W, Cin = x_nhwc.shape
    Cout = w9.shape[-1]
    x_pad = jnp.pad(x_nhwc, ((0, 0), (1, 1), (1, 1), (0, 0)))

    merged = Cin <= 128
    TC = min(Cout, 256)
    TH = H
    while TH * W > 1024 or H % TH or TH % 2:
        TH -= 1
    grid = (B, Cout // TC, H // TH)

    if merged:
        w_arg = w9.reshape(9 * Cin, Cout)
        w_spec = pl.BlockSpec((9 * Cin, TC), lambda b, j, r: (0, j))
    else:
        w_arg = w9
        w_spec = pl.BlockSpec((9, Cin, TC), lambda b, j, r: (0, 0, j))

    Ho = H // 2 if hpool else H
    THo = TH // 2 if hpool else TH
    out_shape = jax.ShapeDtypeStruct((B, Ho, W, Cout), jnp.bfloat16)
    out_spec = pl.BlockSpec((1, THo, W, TC), lambda b, j, r: (b, r, 0, j))

    return pl.pallas_call(
        _make_conv_kernel(merged, hpool),
        grid=grid,
        in_specs=[
            pl.BlockSpec((1, H + 2, W + 2, Cin), lambda b, j, r: (b, 0, 0, 0)),
            w_spec,
            pl.BlockSpec((1, TC), lambda b, j, r: (0, j)),
        ],
        out_specs=out_spec,
        out_shape=out_shape,
        compiler_params=pltpu.CompilerParams(
            dimension_semantics=("parallel", "parallel", "arbitrary")),
    )(x_pad, w_arg.astype(jnp.bfloat16), bias)


def _wpool_kernel(x_ref, o_ref):
    C = o_ref.shape[-1]
    x = x_ref[0]
    o_ref[0] = jnp.maximum(x[..., :C], x[..., C:])


def _wpool(x_bhwc):
    B, Ho, W, C = x_bhwc.shape
    Wo = W // 2
    x_r = x_bhwc.reshape(B, Ho, Wo, 2 * C)
    return pl.pallas_call(
        _wpool_kernel,
        grid=(B,),
        in_specs=[pl.BlockSpec((1, Ho, Wo, 2 * C), lambda b: (b, 0, 0, 0))],
        out_specs=pl.BlockSpec((1, Ho, Wo, C), lambda b: (b, 0, 0, 0)),
        out_shape=jax.ShapeDtypeStruct((B, Ho, Wo, C), jnp.bfloat16),
        compiler_params=pltpu.CompilerParams(dimension_semantics=("parallel",)),
    )(x_r)


def kernel(x, w9_0, b_0, w9_1, b_1, w9_2, b_2, w9_3, b_3, w9_4, b_4,
           w9_5, b_5, w9_6, b_6, w9_7, b_7, w9_8, b_8, w9_9, b_9,
           w9_10, b_10, w9_11, b_11, w9_12, b_12):
    ws = [w9_0, w9_1, w9_2, w9_3, w9_4, w9_5, w9_6, w9_7, w9_8, w9_9,
          w9_10, w9_11, w9_12]
    bs = [b_0, b_1, b_2, b_3, b_4, b_5, b_6, b_7, b_8, b_9, b_10, b_11, b_12]

    h = jnp.transpose(x, (0, 2, 3, 1))
    pooled = []
    pool_after = {1, 3, 6, 9, 12}
    for i in range(13):
        if i == 0:
            h = _stem(h, ws[0], bs[0])
        else:
            h = _conv3x3(h, ws[i], bs[i], i in pool_after)
        if i in pool_after:
            h = _wpool(h)
            pooled.append(h)
    return [jnp.transpose(o, (0, 3, 1, 2)).astype(jnp.float32)
            for o in pooled[1:]]


# speedup vs baseline: 3.3624x; 1.2197x over previous
import jax
import jax.numpy as jnp
from jax.experimental import pallas as pl
from jax.experimental.pallas import tpu as pltpu


def _stem_kernel(p_ref, w_ref, b_ref, o_ref):
    _, TH, W, K = p_ref.shape
    TC = o_ref.shape[-1]
    acc = jnp.dot(p_ref[0].reshape(TH * W, K), w_ref[...],
                  preferred_element_type=jnp.float32)
    acc = jnp.maximum(acc + b_ref[...], 0.0)
    o_ref[0] = acc.reshape(TH, W, TC).astype(o_ref.dtype)


def _stem(x_nhwc, w9, bias):
    B, H, W, _ = x_nhwc.shape
    Cout = w9.shape[-1]
    x_pad = jnp.pad(x_nhwc, ((0, 0), (1, 1), (1, 1), (0, 0)))
    p = jnp.concatenate(
        [x_pad[:, dy:dy + H, dx:dx + W, :] for dy in range(3)
         for dx in range(3)], axis=-1).astype(jnp.bfloat16)
    TH = 16
    grid = (B, H // TH)
    return pl.pallas_call(
        _stem_kernel,
        grid=grid,
        in_specs=[
            pl.BlockSpec((1, TH, W, 27), lambda b, r: (b, r, 0, 0)),
            pl.BlockSpec((27, Cout), lambda b, r: (0, 0)),
            pl.BlockSpec((1, Cout), lambda b, r: (0, 0)),
        ],
        out_specs=pl.BlockSpec((1, TH, W, Cout), lambda b, r: (b, r, 0, 0)),
        out_shape=jax.ShapeDtypeStruct((B, H, W, Cout), jnp.bfloat16),
        compiler_params=pltpu.CompilerParams(
            dimension_semantics=("parallel", "arbitrary")),
    )(p, w9.reshape(27, Cout).astype(jnp.bfloat16), bias)


def _make_conv_kernel(merged, hpool):
    def body(x_ref, w_ref, b_ref, o_ref):
        TH = o_ref.shape[1] * (2 if hpool else 1)
        W = o_ref.shape[2]
        TC = o_ref.shape[3]
        cin = x_ref.shape[-1]
        row0 = pl.multiple_of(pl.program_id(2) * TH, TH)

        if merged:
            taps = []
            for k in range(9):
                dy, dx = k // 3, k % 3
                taps.append(x_ref[0, pl.ds(row0 + dy, TH), pl.ds(dx, W), :])
            patch = jnp.concatenate(taps, axis=-1).reshape(TH * W, 9 * cin)
            acc = jnp.dot(patch, w_ref[...],
                          preferred_element_type=jnp.float32)
        else:
            acc = jnp.zeros((TH * W, TC), dtype=jnp.float32)
            for k in range(9):
                dy, dx = k // 3, k % 3
                patch = x_ref[0, pl.ds(row0 + dy, TH), pl.ds(dx, W), :]
                acc = acc + jnp.dot(patch.reshape(TH * W, cin), w_ref[k],
                                    preferred_element_type=jnp.float32)
        acc = jnp.maximum(acc + b_ref[...], 0.0)
        if hpool:
            m = acc.reshape(TH // 2, 2, W, TC)
            o_ref[0] = jnp.max(m, axis=1).astype(o_ref.dtype)
        else:
            o_ref[0] = acc.reshape(TH, W, TC).astype(o_ref.dtype)
    return body


def _conv3x3(x_nhwc, w9, bias, hpool):
    B, H, W, Cin = x_nhwc.shape
    Cout = w9.shape[-1]
    x_pad = jnp.pad(x_nhwc, ((0, 0), (1, 1), (1, 1), (0, 0)))

    merged = Cin <= 128
    TC = Cout
    TH = H
    while TH * W > 4096 or H % TH or TH % 2:
        TH -= 1
    grid = (B, Cout // TC, H // TH)

    if merged:
        w_arg = w9.reshape(9 * Cin, Cout)
        w_spec = pl.BlockSpec((9 * Cin, TC), lambda b, j, r: (0, j))
    else:
        w_arg = w9
        w_spec = pl.BlockSpec((9, Cin, TC), lambda b, j, r: (0, 0, j))

    Ho = H // 2 if hpool else H
    THo = TH // 2 if hpool else TH
    out_shape = jax.ShapeDtypeStruct((B, Ho, W, Cout), jnp.bfloat16)
    out_spec = pl.BlockSpec((1, THo, W, TC), lambda b, j, r: (b, r, 0, j))

    return pl.pallas_call(
        _make_conv_kernel(merged, hpool),
        grid=grid,
        in_specs=[
            pl.BlockSpec((1, H + 2, W + 2, Cin), lambda b, j, r: (b, 0, 0, 0)),
            w_spec,
            pl.BlockSpec((1, TC), lambda b, j, r: (0, j)),
        ],
        out_specs=out_spec,
        out_shape=out_shape,
        compiler_params=pltpu.CompilerParams(
            dimension_semantics=("parallel", "parallel", "arbitrary")),
    )(x_pad, w_arg.astype(jnp.bfloat16), bias)


def _wpool_kernel(x_ref, o_ref):
    C = o_ref.shape[-1]
    x = x_ref[0]
    o_ref[0] = jnp.maximum(x[..., :C], x[..., C:])


def _wpool(x_bhwc):
    B, Ho, W, C = x_bhwc.shape
    Wo = W // 2
    x_r = x_bhwc.reshape(B, Ho, Wo, 2 * C)
    return pl.pallas_call(
        _wpool_kernel,
        grid=(B,),
        in_specs=[pl.BlockSpec((1, Ho, Wo, 2 * C), lambda b: (b, 0, 0, 0))],
        out_specs=pl.BlockSpec((1, Ho, Wo, C), lambda b: (b, 0, 0, 0)),
        out_shape=jax.ShapeDtypeStruct((B, Ho, Wo, C), jnp.bfloat16),
        compiler_params=pltpu.CompilerParams(dimension_semantics=("parallel",)),
    )(x_r)


def kernel(x, w9_0, b_0, w9_1, b_1, w9_2, b_2, w9_3, b_3, w9_4, b_4,
           w9_5, b_5, w9_6, b_6, w9_7, b_7, w9_8, b_8, w9_9, b_9,
           w9_10, b_10, w9_11, b_11, w9_12, b_12):
    ws = [w9_0, w9_1, w9_2, w9_3, w9_4, w9_5, w9_6, w9_7, w9_8, w9_9,
          w9_10, w9_11, w9_12]
    bs = [b_0, b_1, b_2, b_3, b_4, b_5, b_6, b_7, b_8, b_9, b_10, b_11, b_12]

    h = jnp.transpose(x, (0, 2, 3, 1))
    pooled = []
    pool_after = {1, 3, 6, 9, 12}
    for i in range(13):
        if i == 0:
            h = _stem(h, ws[0], bs[0])
        else:
            h = _conv3x3(h, ws[i], bs[i], i in pool_after)
        if i in pool_after:
            h = _wpool(h)
            pooled.append(h)
    return [jnp.transpose(o, (0, 3, 1, 2)).astype(jnp.float32)
            for o in pooled[1:]]


# speedup vs baseline: 3.4638x; 1.0301x over previous
import jax
import jax.numpy as jnp
from jax.experimental import pallas as pl
from jax.experimental.pallas import tpu as pltpu


def _stem_kernel(p_ref, w_ref, b_ref, o_ref):
    _, TH, WP, K = p_ref.shape
    TC = o_ref.shape[-1]
    nrows = pl.num_programs(1)
    r = pl.program_id(1)
    acc = jnp.dot(p_ref[0].reshape(TH * WP, K), w_ref[...],
                  preferred_element_type=jnp.float32)
    acc = jnp.maximum(acc + b_ref[...], 0.0)
    o_ref[0] = acc.reshape(TH, WP, TC).astype(o_ref.dtype)
    zc = jnp.zeros((TH, TC), o_ref.dtype)
    o_ref[0, :, 0, :] = zc
    o_ref[0, :, WP - 1, :] = zc

    @pl.when(r == 0)
    def _():
        o_ref[0, 0] = jnp.zeros((WP, TC), o_ref.dtype)

    @pl.when(r == nrows - 1)
    def _():
        o_ref[0, TH - 1] = jnp.zeros((WP, TC), o_ref.dtype)


def _stem(x_nhwc, w9, bias):
    B, H, W, _ = x_nhwc.shape
    Cout = w9.shape[-1]
    HP, WP = H + 2, W + 2
    x_pad2 = jnp.pad(x_nhwc, ((0, 0), (2, 2), (2, 2), (0, 0)))
    p = jnp.concatenate(
        [x_pad2[:, dy:dy + HP, dx:dx + WP, :] for dy in range(3)
         for dx in range(3)], axis=-1).astype(jnp.bfloat16)
    TH = HP // 2
    grid = (B, HP // TH)
    return pl.pallas_call(
        _stem_kernel,
        grid=grid,
        in_specs=[
            pl.BlockSpec((1, TH, WP, 27), lambda b, r: (b, r, 0, 0)),
            pl.BlockSpec((27, Cout), lambda b, r: (0, 0)),
            pl.BlockSpec((1, Cout), lambda b, r: (0, 0)),
        ],
        out_specs=pl.BlockSpec((1, TH, WP, Cout), lambda b, r: (b, r, 0, 0)),
        out_shape=jax.ShapeDtypeStruct((B, HP, WP, Cout), jnp.bfloat16),
        compiler_params=pltpu.CompilerParams(
            dimension_semantics=("parallel", "arbitrary")),
    )(p, w9.reshape(27, Cout).astype(jnp.bfloat16), bias)


def _make_conv_kernel(merged, hpool):
    def body(x_ref, w_ref, b_ref, o_ref):
        TH = o_ref.shape[1] * (2 if hpool else 1)
        W = o_ref.shape[2]
        TC = o_ref.shape[3]
        cin = x_ref.shape[-1]
        row0 = pl.multiple_of(pl.program_id(2) * TH, TH)

        xs = [x_ref[0, pl.ds(row0, TH + 2), pl.ds(dx, W), :] for dx in range(3)]
        if merged:
            taps = []
            for k in range(9):
                dy, dx = k // 3, k % 3
                taps.append(xs[dx][dy:dy + TH])
            patch = jnp.concatenate(taps, axis=-1).reshape(TH * W, 9 * cin)
            acc = jnp.dot(patch, w_ref[...],
                          preferred_element_type=jnp.float32)
        else:
            acc = jnp.zeros((TH * W, TC), dtype=jnp.float32)
            for k in range(9):
                dy, dx = k // 3, k % 3
                acc = acc + jnp.dot(xs[dx][dy:dy + TH].reshape(TH * W, cin),
                                    w_ref[k],
                                    preferred_element_type=jnp.float32)
        acc = jnp.maximum(acc + b_ref[...], 0.0)
        if hpool:
            m = acc.reshape(TH // 2, 2, W, TC)
            o_ref[0] = jnp.max(m, axis=1).astype(o_ref.dtype)
        else:
            o_ref[0] = acc.reshape(TH, W, TC).astype(o_ref.dtype)
    return body


def _conv3x3(x_pad, w9, bias, hpool):
    B, HP, WP, Cin = x_pad.shape
    H, W = HP - 2, WP - 2
    Cout = w9.shape[-1]

    merged = Cin <= 128
    TC = Cout
    TH = H
    while TH * W > 4096 or H % TH or TH % 2:
        TH -= 1
    grid = (B, Cout // TC, H // TH)

    if merged:
        w_arg = w9.reshape(9 * Cin, Cout)
        w_spec = pl.BlockSpec((9 * Cin, TC), lambda b, j, r: (0, j))
    else:
        w_arg = w9
        w_spec = pl.BlockSpec((9, Cin, TC), lambda b, j, r: (0, 0, j))

    Ho = H // 2 if hpool else H
    THo = TH // 2 if hpool else TH
    out_shape = jax.ShapeDtypeStruct((B, Ho, W, Cout), jnp.bfloat16)
    out_spec = pl.BlockSpec((1, THo, W, TC), lambda b, j, r: (b, r, 0, j))

    return pl.pallas_call(
        _make_conv_kernel(merged, hpool),
        grid=grid,
        in_specs=[
            pl.BlockSpec((1, H + 2, W + 2, Cin), lambda b, j, r: (b, 0, 0, 0)),
            w_spec,
            pl.BlockSpec((1, TC), lambda b, j, r: (0, j)),
        ],
        out_specs=out_spec,
        out_shape=out_shape,
        compiler_params=pltpu.CompilerParams(
            dimension_semantics=("parallel", "parallel", "arbitrary")),
    )(x_pad, w_arg.astype(jnp.bfloat16), bias)


def _wpool_kernel(x_ref, o_ref):
    C = o_ref.shape[-1]
    x = x_ref[0]
    o_ref[0] = jnp.maximum(x[..., :C], x[..., C:])


def _wpool(x_bhwc):
    B, Ho, W, C = x_bhwc.shape
    Wo = W // 2
    x_r = x_bhwc.reshape(B, Ho, Wo, 2 * C)
    return pl.pallas_call(
        _wpool_kernel,
        grid=(B,),
        in_specs=[pl.BlockSpec((1, Ho, Wo, 2 * C), lambda b: (b, 0, 0, 0))],
        out_specs=pl.BlockSpec((1, Ho, Wo, C), lambda b: (b, 0, 0, 0)),
        out_shape=jax.ShapeDtypeStruct((B, Ho, Wo, C), jnp.bfloat16),
        compiler_params=pltpu.CompilerParams(dimension_semantics=("parallel",)),
    )(x_r)


def kernel(x, w9_0, b_0, w9_1, b_1, w9_2, b_2, w9_3, b_3, w9_4, b_4,
           w9_5, b_5, w9_6, b_6, w9_7, b_7, w9_8, b_8, w9_9, b_9,
           w9_10, b_10, w9_11, b_11, w9_12, b_12):
    ws = [w9_0, w9_1, w9_2, w9_3, w9_4, w9_5, w9_6, w9_7, w9_8, w9_9,
          w9_10, w9_11, w9_12]
    bs = [b_0, b_1, b_2, b_3, b_4, b_5, b_6, b_7, b_8, b_9, b_10, b_11, b_12]

    h = jnp.transpose(x, (0, 2, 3, 1))
    pooled = []
    pool_after = {1, 3, 6, 9, 12}
    hp = _stem(h, ws[0], bs[0])
    for i in range(1, 13):
        out = _conv3x3(hp, ws[i], bs[i], i in pool_after)
        if i in pool_after:
            out = _wpool(out)
            pooled.append(out)
        if i < 12:
            hp = jnp.pad(out, ((0, 0), (1, 1), (1, 1), (0, 0)))
    return [jnp.transpose(o, (0, 3, 1, 2)).astype(jnp.float32)
            for o in pooled[1:]]


# speedup vs baseline: 3.6961x; 1.0671x over previous
import jax
import jax.numpy as jnp
from jax.experimental import pallas as pl
from jax.experimental.pallas import tpu as pltpu


def _stem_kernel(p_ref, w_ref, b_ref, o_ref):
    _, TH, WS, K = p_ref.shape
    TC = o_ref.shape[-1]
    nrows = pl.num_programs(1)
    r = pl.program_id(1)
    acc = jnp.dot(p_ref[0].reshape(TH * WS, K), w_ref[...],
                  preferred_element_type=jnp.float32)
    acc = jnp.maximum(acc + b_ref[...], 0.0)
    o_ref[0] = acc.reshape(TH, WS, TC).astype(o_ref.dtype)
    zc = jnp.zeros((TH, TC), o_ref.dtype)
    o_ref[0, :, 0, :] = zc
    o_ref[0, :, WS - 7, :] = zc

    @pl.when(r == 0)
    def _():
        o_ref[0, 0] = jnp.zeros((WS, TC), o_ref.dtype)

    @pl.when(r == nrows - 1)
    def _():
        o_ref[0, TH - 1] = jnp.zeros((WS, TC), o_ref.dtype)


def _stem(x_nhwc, w9, bias):
    B, H, W, _ = x_nhwc.shape
    Cout = w9.shape[-1]
    HP, WS = H + 2, W + 8
    x_pad2 = jnp.pad(x_nhwc, ((0, 0), (2, 2), (2, 8), (0, 0)))
    p = jnp.concatenate(
        [x_pad2[:, dy:dy + HP, dx:dx + WS, :] for dy in range(3)
         for dx in range(3)], axis=-1).astype(jnp.bfloat16)
    TH = HP // 2
    grid = (B, HP // TH)
    return pl.pallas_call(
        _stem_kernel,
        grid=grid,
        in_specs=[
            pl.BlockSpec((1, TH, WS, 27), lambda b, r: (b, r, 0, 0)),
            pl.BlockSpec((27, Cout), lambda b, r: (0, 0)),
            pl.BlockSpec((1, Cout), lambda b, r: (0, 0)),
        ],
        out_specs=pl.BlockSpec((1, TH, WS, Cout), lambda b, r: (b, r, 0, 0)),
        out_shape=jax.ShapeDtypeStruct((B, HP, WS, Cout), jnp.bfloat16),
        compiler_params=pltpu.CompilerParams(
            dimension_semantics=("parallel", "arbitrary")),
    )(p, w9.reshape(27, Cout).astype(jnp.bfloat16), bias)


def _make_conv_kernel(hpool, W, Wc):
    def body(x_ref, w_ref, b_ref, o_ref):
        THo = o_ref.shape[1]
        TH = THo * (2 if hpool else 1)
        TC = o_ref.shape[3]
        cin = x_ref.shape[-1]
        row0 = pl.multiple_of(pl.program_id(1) * TH, TH)

        xs = [x_ref[0, pl.ds(row0, TH + 2), pl.ds(dx, Wc), :]
              for dx in range(3)]
        taps = [xs[dx][dy:dy + TH] for dy in range(3) for dx in range(3)]
        patch = jnp.concatenate(taps, axis=-1).reshape(TH * Wc, 9 * cin)
        acc = jnp.dot(patch, w_ref[...], preferred_element_type=jnp.float32)
        acc = jnp.maximum(acc + b_ref[...], 0.0)
        if hpool:
            res = jnp.max(acc.reshape(THo, 2, Wc, TC), axis=1)
        else:
            res = acc.reshape(TH, Wc, TC)
        o_ref[0] = res.astype(o_ref.dtype)
        if Wc > W:
            o_ref[0, :, pl.ds(W, Wc - W), :] = jnp.zeros(
                (THo, Wc - W, TC), o_ref.dtype)
    return body


def _conv3x3(x_pad, w9, bias, W, hpool):
    B, HPin, WPin, Cin = x_pad.shape
    H = HPin - 2
    Wc = -(-W // 8) * 8
    Cout = w9.shape[-1]

    cap = min(4096, (8 * 1024 * 1024) // (9 * Cin * 2))
    TH = H
    while TH * Wc > cap or H % TH or TH % 2:
        TH -= 1
    THo = TH // 2 if hpool else TH
    Ho = H // 2 if hpool else H

    return pl.pallas_call(
        _make_conv_kernel(hpool, W, Wc),
        grid=(B, H // TH),
        in_specs=[
            pl.BlockSpec((1, HPin, WPin, Cin), lambda b, r: (b, 0, 0, 0)),
            pl.BlockSpec((9 * Cin, Cout), lambda b, r: (0, 0)),
            pl.BlockSpec((1, Cout), lambda b, r: (0, 0)),
        ],
        out_specs=pl.BlockSpec((1, THo, Wc, Cout), lambda b, r: (b, r, 0, 0)),
        out_shape=jax.ShapeDtypeStruct((B, Ho, Wc, Cout), jnp.bfloat16),
        compiler_params=pltpu.CompilerParams(
            dimension_semantics=("parallel", "arbitrary")),
    )(x_pad, w9.reshape(9 * Cin, Cout).astype(jnp.bfloat16), bias)


def _wpool_kernel(x_ref, o_ref):
    C = o_ref.shape[-1]
    x = x_ref[0]
    o_ref[0] = jnp.maximum(x[..., :C], x[..., C:])


def _wpool(x_bhwc):
    B, Ho, Wb, C = x_bhwc.shape
    Wo = Wb // 2
    x_r = x_bhwc.reshape(B, Ho, Wo, 2 * C)
    return pl.pallas_call(
        _wpool_kernel,
        grid=(B,),
        in_specs=[pl.BlockSpec((1, Ho, Wo, 2 * C), lambda b: (b, 0, 0, 0))],
        out_specs=pl.BlockSpec((1, Ho, Wo, C), lambda b: (b, 0, 0, 0)),
        out_shape=jax.ShapeDtypeStruct((B, Ho, Wo, C), jnp.bfloat16),
        compiler_params=pltpu.CompilerParams(dimension_semantics=("parallel",)),
    )(x_r)


def kernel(x, w9_0, b_0, w9_1, b_1, w9_2, b_2, w9_3, b_3, w9_4, b_4,
           w9_5, b_5, w9_6, b_6, w9_7, b_7, w9_8, b_8, w9_9, b_9,
           w9_10, b_10, w9_11, b_11, w9_12, b_12):
    ws = [w9_0, w9_1, w9_2, w9_3, w9_4, w9_5, w9_6, w9_7, w9_8, w9_9,
          w9_10, w9_11, w9_12]
    bs = [b_0, b_1, b_2, b_3, b_4, b_5, b_6, b_7, b_8, b_9, b_10, b_11, b_12]

    h = jnp.transpose(x, (0, 2, 3, 1))
    pooled = []
    pool_after = {1, 3, 6, 9, 12}
    W = h.shape[2]
    hp = _stem(h, ws[0], bs[0])
    for i in range(1, 13):
        out = _conv3x3(hp, ws[i], bs[i], W, i in pool_after)
        if i in pool_after:
            out = _wpool(out)
            W //= 2
            pooled.append(out)
        if i < 12:
            right = (-(-W // 8) * 8 + 2) - (out.shape[2] + 1)
            hp = jnp.pad(out, ((0, 0), (1, 1), (1, right), (0, 0)))
    return [jnp.transpose(o[:, :, :W0, :], (0, 3, 1, 2)).astype(jnp.float32)
            for o, W0 in zip(pooled[1:], (56, 28, 14, 7))]


# speedup vs baseline: 3.8370x; 1.0381x over previous
import jax
import jax.numpy as jnp
from jax.experimental import pallas as pl
from jax.experimental.pallas import tpu as pltpu


def _stem_kernel(p_ref, w_ref, b_ref, o_ref):
    _, TH, WS, K = p_ref.shape
    TC = o_ref.shape[-1]
    nrows = pl.num_programs(1)
    r = pl.program_id(1)
    acc = jnp.dot(p_ref[0].reshape(TH * WS, K), w_ref[...],
                  preferred_element_type=jnp.float32)
    acc = jnp.maximum(acc + b_ref[...], 0.0)
    o_ref[0] = acc.reshape(TH, WS, TC).astype(o_ref.dtype)
    zc = jnp.zeros((TH, TC), o_ref.dtype)
    o_ref[0, :, 0, :] = zc
    o_ref[0, :, WS - 7, :] = zc

    @pl.when(r == 0)
    def _():
        o_ref[0, 0] = jnp.zeros((WS, TC), o_ref.dtype)

    @pl.when(r == nrows - 1)
    def _():
        o_ref[0, TH - 1] = jnp.zeros((WS, TC), o_ref.dtype)


def _stem(x_nhwc, w9, bias):
    B, H, W, _ = x_nhwc.shape
    Cout = w9.shape[-1]
    HP, WS = H + 2, W + 8
    x_pad2 = jnp.pad(x_nhwc, ((0, 0), (2, 2), (2, 8), (0, 0)))
    p = jnp.concatenate(
        [x_pad2[:, dy:dy + HP, dx:dx + WS, :] for dy in range(3)
         for dx in range(3)], axis=-1).astype(jnp.bfloat16)
    TH = HP // 2
    grid = (B, HP // TH)
    return pl.pallas_call(
        _stem_kernel,
        grid=grid,
        in_specs=[
            pl.BlockSpec((1, TH, WS, 27), lambda b, r: (b, r, 0, 0)),
            pl.BlockSpec((27, Cout), lambda b, r: (0, 0)),
            pl.BlockSpec((1, Cout), lambda b, r: (0, 0)),
        ],
        out_specs=pl.BlockSpec((1, TH, WS, Cout), lambda b, r: (b, r, 0, 0)),
        out_shape=jax.ShapeDtypeStruct((B, HP, WS, Cout), jnp.bfloat16),
        compiler_params=pltpu.CompilerParams(
            dimension_semantics=("parallel", "arbitrary")),
    )(p, w9.reshape(27, Cout).astype(jnp.bfloat16), bias)


def _make_conv_kernel(hpool, W, Wc):
    def body(x_ref, w_ref, b_ref, o_ref):
        THo = o_ref.shape[1]
        TH = THo * (2 if hpool else 1)
        TC = o_ref.shape[3]
        cin = x_ref.shape[-1]
        row0 = pl.multiple_of(pl.program_id(1) * TH, TH)

        xs = [x_ref[0, pl.ds(row0, TH + 2), pl.ds(dx, Wc), :]
              for dx in range(3)]
        taps = [xs[dx][dy:dy + TH] for dy in range(3) for dx in range(3)]
        patch = jnp.concatenate(taps, axis=-1).reshape(TH * Wc, 9 * cin)
        acc = jnp.dot(patch, w_ref[...], preferred_element_type=jnp.float32)
        acc = jnp.maximum(acc + b_ref[...], 0.0)
        if hpool:
            res = jnp.max(acc.reshape(THo, 2, Wc, TC), axis=1)
        else:
            res = acc.reshape(TH, Wc, TC)
        o_ref[0] = res.astype(o_ref.dtype)
        if Wc > W:
            o_ref[0, :, pl.ds(W, Wc - W), :] = jnp.zeros(
                (THo, Wc - W, TC), o_ref.dtype)
    return body


def _conv3x3(x_pad, w9, bias, W, hpool):
    B, HPin, WPin, Cin = x_pad.shape
    H = HPin - 2
    Wc = -(-W // 8) * 8
    Cout = w9.shape[-1]

    cap = min(8192, (16 * 1024 * 1024) // (9 * Cin * 2))
    TH = H
    while TH * Wc > cap or H % TH or TH % 2:
        TH -= 1
    THo = TH // 2 if hpool else TH
    Ho = H // 2 if hpool else H

    return pl.pallas_call(
        _make_conv_kernel(hpool, W, Wc),
        grid=(B, H // TH),
        in_specs=[
            pl.BlockSpec((1, HPin, WPin, Cin), lambda b, r: (b, 0, 0, 0)),
            pl.BlockSpec((9 * Cin, Cout), lambda b, r: (0, 0)),
            pl.BlockSpec((1, Cout), lambda b, r: (0, 0)),
        ],
        out_specs=pl.BlockSpec((1, THo, Wc, Cout), lambda b, r: (b, r, 0, 0)),
        out_shape=jax.ShapeDtypeStruct((B, Ho, Wc, Cout), jnp.bfloat16),
        compiler_params=pltpu.CompilerParams(
            dimension_semantics=("parallel", "arbitrary")),
    )(x_pad, w9.reshape(9 * Cin, Cout).astype(jnp.bfloat16), bias)


def _wpool_kernel(x_ref, o_ref):
    C = o_ref.shape[-1]
    x = x_ref[0]
    o_ref[0] = jnp.maximum(x[..., :C], x[..., C:])


def _wpool(x_bhwc):
    B, Ho, Wb, C = x_bhwc.shape
    Wo = Wb // 2
    x_r = x_bhwc.reshape(B, Ho, Wo, 2 * C)
    return pl.pallas_call(
        _wpool_kernel,
        grid=(B,),
        in_specs=[pl.BlockSpec((1, Ho, Wo, 2 * C), lambda b: (b, 0, 0, 0))],
        out_specs=pl.BlockSpec((1, Ho, Wo, C), lambda b: (b, 0, 0, 0)),
        out_shape=jax.ShapeDtypeStruct((B, Ho, Wo, C), jnp.bfloat16),
        compiler_params=pltpu.CompilerParams(dimension_semantics=("parallel",)),
    )(x_r)


def kernel(x, w9_0, b_0, w9_1, b_1, w9_2, b_2, w9_3, b_3, w9_4, b_4,
           w9_5, b_5, w9_6, b_6, w9_7, b_7, w9_8, b_8, w9_9, b_9,
           w9_10, b_10, w9_11, b_11, w9_12, b_12):
    ws = [w9_0, w9_1, w9_2, w9_3, w9_4, w9_5, w9_6, w9_7, w9_8, w9_9,
          w9_10, w9_11, w9_12]
    bs = [b_0, b_1, b_2, b_3, b_4, b_5, b_6, b_7, b_8, b_9, b_10, b_11, b_12]

    h = jnp.transpose(x, (0, 2, 3, 1))
    pooled = []
    pool_after = {1, 3, 6, 9, 12}
    W = h.shape[2]
    hp = _stem(h, ws[0], bs[0])
    for i in range(1, 13):
        out = _conv3x3(hp, ws[i], bs[i], W, i in pool_after)
        if i in pool_after:
            out = _wpool(out)
            W //= 2
            pooled.append(out)
        if i < 12:
            right = (-(-W // 8) * 8 + 2) - (out.shape[2] + 1)
            hp = jnp.pad(out, ((0, 0), (1, 1), (1, right), (0, 0)))
    return [jnp.transpose(o[:, :, :W0, :], (0, 3, 1, 2)).astype(jnp.float32)
            for o, W0 in zip(pooled[1:], (56, 28, 14, 7))]


# speedup vs baseline: 4.0350x; 1.0516x over previous
import jax
import jax.numpy as jnp
from jax.experimental import pallas as pl
from jax.experimental.pallas import tpu as pltpu


def _stem_kernel(p_ref, w_ref, b_ref, o_ref):
    _, TH, WS, K = p_ref.shape
    TC = o_ref.shape[-1]
    nrows = pl.num_programs(1)
    r = pl.program_id(1)
    acc = jnp.dot(p_ref[0].reshape(TH * WS, K), w_ref[...],
                  preferred_element_type=jnp.float32)
    acc = jnp.maximum(acc + b_ref[...], 0.0)
    o_ref[0] = acc.reshape(TH, WS, TC).astype(o_ref.dtype)
    zc = jnp.zeros((TH, TC), o_ref.dtype)
    o_ref[0, :, 0, :] = zc
    o_ref[0, :, WS - 7, :] = zc

    @pl.when(r == 0)
    def _():
        o_ref[0, 0] = jnp.zeros((WS, TC), o_ref.dtype)

    @pl.when(r == nrows - 1)
    def _():
        o_ref[0, TH - 1] = jnp.zeros((WS, TC), o_ref.dtype)


def _stem(x_nhwc, w9, bias):
    B, H, W, _ = x_nhwc.shape
    Cout = w9.shape[-1]
    HP, WS = H + 2, W + 8
    x_pad2 = jnp.pad(x_nhwc, ((0, 0), (2, 2), (2, 8), (0, 0)))
    p = jnp.concatenate(
        [x_pad2[:, dy:dy + HP, dx:dx + WS, :] for dy in range(3)
         for dx in range(3)], axis=-1).astype(jnp.bfloat16)
    TH = HP // 2
    grid = (B, HP // TH)
    return pl.pallas_call(
        _stem_kernel,
        grid=grid,
        in_specs=[
            pl.BlockSpec((1, TH, WS, 27), lambda b, r: (b, r, 0, 0)),
            pl.BlockSpec((27, Cout), lambda b, r: (0, 0)),
            pl.BlockSpec((1, Cout), lambda b, r: (0, 0)),
        ],
        out_specs=pl.BlockSpec((1, TH, WS, Cout), lambda b, r: (b, r, 0, 0)),
        out_shape=jax.ShapeDtypeStruct((B, HP, WS, Cout), jnp.bfloat16),
        compiler_params=pltpu.CompilerParams(
            dimension_semantics=("parallel", "arbitrary")),
    )(p, w9.reshape(27, Cout).astype(jnp.bfloat16), bias)


def _make_conv_kernel(hpool, W, Wc, pad_out):
    def body(x_ref, w_ref, b_ref, o_ref):
        THo = o_ref.shape[1] - (2 if pad_out else 0)
        TH = THo * (2 if hpool else 1)
        TC = o_ref.shape[3]
        cin = x_ref.shape[-1]
        row0 = pl.multiple_of(pl.program_id(1) * TH, TH)

        xs = [x_ref[0, pl.ds(row0, TH + 2), pl.ds(dx, Wc), :]
              for dx in range(3)]
        taps = [xs[dx][dy:dy + TH] for dy in range(3) for dx in range(3)]
        patch = jnp.concatenate(taps, axis=-1).reshape(TH * Wc, 9 * cin)
        acc = jnp.dot(patch, w_ref[...], preferred_element_type=jnp.float32)
        acc = jnp.maximum(acc + b_ref[...], 0.0)
        if hpool:
            res = jnp.max(acc.reshape(THo, 2, Wc, TC), axis=1)
        else:
            res = acc.reshape(TH, Wc, TC)
        if pad_out:
            o_ref[0, pl.ds(1, THo), pl.ds(1, Wc), :] = res.astype(o_ref.dtype)
            WP = Wc + 2
            o_ref[0, 0] = jnp.zeros((WP, TC), o_ref.dtype)
            o_ref[0, THo + 1] = jnp.zeros((WP, TC), o_ref.dtype)
            zc = jnp.zeros((THo + 2, 1, TC), o_ref.dtype)
            o_ref[0, :, pl.ds(0, 1), :] = zc
            o_ref[0, :, pl.ds(W + 1, WP - 1 - W), :] = jnp.zeros(
                (THo + 2, WP - 1 - W, TC), o_ref.dtype)
        else:
            o_ref[0] = res.astype(o_ref.dtype)
            if Wc > W:
                o_ref[0, :, pl.ds(W, Wc - W), :] = jnp.zeros(
                    (THo, Wc - W, TC), o_ref.dtype)
    return body


def _conv3x3(x_pad, w9, bias, W, hpool, pad_out=False):
    B, HPin, WPin, Cin = x_pad.shape
    H = HPin - 2
    Wc = -(-W // 8) * 8
    Cout = w9.shape[-1]

    cap = min(8192, (16 * 1024 * 1024) // (9 * Cin * 2))
    TH = H
    while TH * Wc > cap or H % TH or TH % 2:
        TH -= 1
    THo = TH // 2 if hpool else TH
    Ho = H // 2 if hpool else H
    pad_out = pad_out and TH == H and not hpool
    if pad_out:
        out_shape = jax.ShapeDtypeStruct((B, Ho + 2, Wc + 2, Cout),
                                         jnp.bfloat16)
        out_spec = pl.BlockSpec((1, Ho + 2, Wc + 2, Cout),
                                lambda b, r: (b, 0, 0, 0))
    else:
        out_shape = jax.ShapeDtypeStruct((B, Ho, Wc, Cout), jnp.bfloat16)
        out_spec = pl.BlockSpec((1, THo, Wc, Cout), lambda b, r: (b, r, 0, 0))

    out = pl.pallas_call(
        _make_conv_kernel(hpool, W, Wc, pad_out),
        grid=(B, H // TH),
        in_specs=[
            pl.BlockSpec((1, HPin, WPin, Cin), lambda b, r: (b, 0, 0, 0)),
            pl.BlockSpec((9 * Cin, Cout), lambda b, r: (0, 0)),
            pl.BlockSpec((1, Cout), lambda b, r: (0, 0)),
        ],
        out_specs=out_spec,
        out_shape=out_shape,
        compiler_params=pltpu.CompilerParams(
            dimension_semantics=("parallel", "arbitrary")),
    )(x_pad, w9.reshape(9 * Cin, Cout).astype(jnp.bfloat16), bias)
    return out, pad_out


def _wpool_kernel(x_ref, o_ref):
    C = o_ref.shape[-1]
    x = x_ref[0]
    o_ref[0] = jnp.maximum(x[..., :C], x[..., C:])


def _wpool(x_bhwc):
    B, Ho, Wb, C = x_bhwc.shape
    Wo = Wb // 2
    x_r = x_bhwc.reshape(B, Ho, Wo, 2 * C)
    return pl.pallas_call(
        _wpool_kernel,
        grid=(B,),
        in_specs=[pl.BlockSpec((1, Ho, Wo, 2 * C), lambda b: (b, 0, 0, 0))],
        out_specs=pl.BlockSpec((1, Ho, Wo, C), lambda b: (b, 0, 0, 0)),
        out_shape=jax.ShapeDtypeStruct((B, Ho, Wo, C), jnp.bfloat16),
        compiler_params=pltpu.CompilerParams(dimension_semantics=("parallel",)),
    )(x_r)


def kernel(x, w9_0, b_0, w9_1, b_1, w9_2, b_2, w9_3, b_3, w9_4, b_4,
           w9_5, b_5, w9_6, b_6, w9_7, b_7, w9_8, b_8, w9_9, b_9,
           w9_10, b_10, w9_11, b_11, w9_12, b_12):
    ws = [w9_0, w9_1, w9_2, w9_3, w9_4, w9_5, w9_6, w9_7, w9_8, w9_9,
          w9_10, w9_11, w9_12]
    bs = [b_0, b_1, b_2, b_3, b_4, b_5, b_6, b_7, b_8, b_9, b_10, b_11, b_12]

    h = jnp.transpose(x, (0, 2, 3, 1))
    pooled = []
    pool_after = {1, 3, 6, 9, 12}
    W = h.shape[2]
    hp = _stem(h, ws[0], bs[0])
    for i in range(1, 13):
        out, padded = _conv3x3(hp, ws[i], bs[i], W, i in pool_after,
                               pad_out=(i < 12 and i not in pool_after))
        if i in pool_after:
            out = _wpool(out)
            W //= 2
            pooled.append(out)
        if i < 12:
            if padded:
                hp = out
            else:
                right = (-(-W // 8) * 8 + 2) - (out.shape[2] + 1)
                hp = jnp.pad(out, ((0, 0), (1, 1), (1, right), (0, 0)))
    return [jnp.transpose(o[:, :, :W0, :], (0, 3, 1, 2)).astype(jnp.float32)
            for o, W0 in zip(pooled[1:], (56, 28, 14, 7))]
